# baseline (speedup 1.0000x reference)
import functools

import jax
import jax.numpy as jnp
from jax import lax
from jax.experimental import pallas as pl
from jax.experimental.pallas import tpu as pltpu
from jax.experimental.pallas import tpu_sc as plsc

NC, NS = 2, 16
NW = NC * NS
HP = 56
GATHER = 112
SEQ_PER_CHUNK = 8
K = 4
CHUNK = K * GATHER


def _body(table, idx, out, idx_v, rows_v, gsem, ssem):
    wid = lax.axis_index("s") * NC + lax.axis_index("c")
    n_seq = out.shape[0]
    H = out.shape[1]
    seq_per_w = n_seq // NW
    chunks = seq_per_w // SEQ_PER_CHUNK
    base_seq = wid * seq_per_w

    pltpu.sync_copy(idx.at[pl.ds(base_seq * HP, seq_per_w * HP)], idx_v)

    def fire_gathers(c, b):
        for j in range(K):
            pltpu.async_copy(
                table.at[idx_v.at[pl.ds(c * CHUNK + j * GATHER, GATHER)]],
                rows_v.at[b, pl.ds(j * GATHER, GATHER)],
                gsem.at[b],
            )

    def drain_gathers(b):
        pltpu.make_async_copy(
            out.at[0], rows_v.at[b, pl.ds(0, CHUNK)], gsem.at[b]
        ).wait()

    def fire_stores(c, b):
        for s in range(SEQ_PER_CHUNK):
            pltpu.async_copy(
                rows_v.at[b, pl.ds(s * HP, H)],
                out.at[base_seq + c * SEQ_PER_CHUNK + s],
                ssem.at[b],
            )

    def drain_stores(b):
        for s in range(SEQ_PER_CHUNK):
            pltpu.make_async_copy(
                rows_v.at[b, pl.ds(s * HP, H)], out.at[0], ssem.at[b]
            ).wait()

    fire_gathers(0, 0)

    @pl.loop(0, chunks - 1)
    def _pipe(c):
        b = c % 2
        nb = 1 - b
        @pl.when(c >= 1)
        def _():
            drain_stores(nb)
        fire_gathers(c + 1, nb)
        drain_gathers(b)
        fire_stores(c, b)

    last = chunks - 1
    lb = last % 2
    drain_gathers(lb)
    fire_stores(last, lb)
    drain_stores(lb)
    drain_stores(1 - lb)


def kernel(x, weight):
    B, H = x.shape
    V, D = weight.shape
    xp = jnp.pad(x.astype(jnp.int32), ((0, 0), (0, HP - H)))
    idx_flat = xp.reshape(B * HP)

    mesh = plsc.VectorSubcoreMesh(
        core_axis_name="c", subcore_axis_name="s",
        num_cores=NC, num_subcores=NS,
    )
    run = pl.kernel(
        _body,
        out_type=jax.ShapeDtypeStruct((B, H, D), jnp.float32),
        mesh=mesh,
        scratch_types=[
            pltpu.VMEM((B * HP // NW,), jnp.int32),
            pltpu.VMEM((2, CHUNK, D), jnp.float32),
            pltpu.SemaphoreType.DMA((2,)),
            pltpu.SemaphoreType.DMA((2,)),
        ],
        compiler_params=pltpu.CompilerParams(use_tc_tiling_on_sc=False),
    )
    return run(weight, idx_flat)
